# speedup vs baseline: 1.2132x; 1.2132x over previous
import functools

import jax
import jax.numpy as jnp
from jax import lax
from jax.experimental import pallas as pl
from jax.experimental.pallas import tpu as pltpu

B = 2048
D = 256
R = 256
NB = B // R
GAMMA = 0.8
EPS = 1e-14
RHO = 8.0
TAU_INIT = 0.01
BETA_U = 0.5
GRAD_CLIP = 5.0
ETA_INIT = 1e-05


def _side_body(feat_ref, otherT_ref, tau_ref, s_ref, b_ref, e0_ref,
               loss_ref, twsum_ref, twmax_ref, twmin_ref, tausum_ref):
    i = pl.program_id(0)
    feat = feat_ref[...]
    otherT = otherT_ref[...]
    S = jnp.dot(feat, otherT, preferred_element_type=jnp.float32,
                precision=lax.Precision.HIGHEST)
    row = lax.broadcasted_iota(jnp.int32, (R, B), 0)
    col = lax.broadcasted_iota(jnp.int32, (R, B), 1)
    is_diag = col == (i * R + row)
    d = jnp.sum(jnp.where(is_diag, S, 0.0), axis=1)
    tau = tau_ref[0, 0, :]
    s_old = s_ref[0, 0, :]
    b_old = b_ref[0, 0, :]
    diffs = S - d[:, None]
    idt = diffs / tau[:, None]
    b_new = jnp.maximum(b_old, jnp.max(idt, axis=1))
    E = jnp.exp(idt - b_new[:, None])
    E = jnp.where(is_diag, 0.0, E)
    g = jnp.sum(E, axis=1)
    ema = (1.0 - GAMMA) * s_old * jnp.exp(b_old - b_new) + GAMMA * g
    e0 = e0_ref[0, 0]
    sI = e0 * g + (1.0 - e0) * ema
    sIc = jnp.maximum(sI, EPS)
    w = E / sIc[:, None]
    loss_rows = jnp.sum(w * diffs, axis=1)
    wid_rows = jnp.sum(w * idt, axis=1)
    tw = jnp.log(sIc / (B - 1)) + b_new + RHO - wid_rows
    tw = jnp.clip(tw, -GRAD_CLIP, GRAD_CLIP)

    blk_loss = jnp.sum(loss_rows)
    blk_twsum = jnp.sum(tw)
    blk_twmax = jnp.max(tw)
    blk_twmin = jnp.min(tw)
    blk_tau = jnp.sum(tau)

    @pl.when(i == 0)
    def _init():
        loss_ref[0, 0] = blk_loss
        twsum_ref[0, 0] = blk_twsum
        twmax_ref[0, 0] = blk_twmax
        twmin_ref[0, 0] = blk_twmin
        tausum_ref[0, 0] = blk_tau

    @pl.when(i != 0)
    def _acc():
        loss_ref[0, 0] += blk_loss
        twsum_ref[0, 0] += blk_twsum
        twmax_ref[0, 0] = jnp.maximum(twmax_ref[0, 0], blk_twmax)
        twmin_ref[0, 0] = jnp.minimum(twmin_ref[0, 0], blk_twmin)
        tausum_ref[0, 0] += blk_tau


_scal = jax.ShapeDtypeStruct((1, 1), jnp.float32)


def _side(feat, otherT, tau_g, s_g, b_g, e0, interpret=False):
    tau3 = tau_g.reshape(NB, 1, R)
    s3 = s_g.reshape(NB, 1, R)
    b3 = b_g.reshape(NB, 1, R)
    smem = pltpu.MemorySpace.SMEM
    out = pl.pallas_call(
        _side_body,
        grid=(NB,),
        in_specs=[
            pl.BlockSpec((R, D), lambda i: (i, 0)),
            pl.BlockSpec((D, B), lambda i: (0, 0)),
            pl.BlockSpec((1, 1, R), lambda i: (i, 0, 0)),
            pl.BlockSpec((1, 1, R), lambda i: (i, 0, 0)),
            pl.BlockSpec((1, 1, R), lambda i: (i, 0, 0)),
            pl.BlockSpec(memory_space=smem),
        ],
        out_specs=[pl.BlockSpec((1, 1), lambda i: (0, 0), memory_space=smem)] * 5,
        out_shape=[_scal] * 5,
        interpret=interpret,
    )(feat, otherT, tau3, s3, b3, e0)
    return out


def kernel(image_features, text_features, image_ids, text_ids, epoch, max_epoch,
           s_I, s_T, b_I, b_T, u_I, u_T, tau_I, tau_T, mask_neg):
    tau_i = tau_I[image_ids]
    s_i = s_I[image_ids]
    b_i = b_I[image_ids]
    tau_t = tau_T[text_ids]
    s_t = s_T[text_ids]
    b_t = b_T[text_ids]

    e0 = (jnp.asarray(epoch) == 0).astype(jnp.float32).reshape(1, 1)
    txtT = text_features.T
    imgT = image_features.T

    li, twi_s, twi_mx, twi_mn, tau_si = _side(image_features, txtT, tau_i, s_i, b_i, e0)
    lt, twt_s, _twt_mx, _twt_mn, tau_st = _side(text_features, imgT, tau_t, s_t, b_t, e0)

    invB = jnp.float32(1.0 / B)
    total_loss = (li[0, 0] + lt[0, 0]) * invB
    return (total_loss,
            tau_si[0, 0] * invB,
            tau_st[0, 0] * invB,
            jnp.float32(ETA_INIT),
            twi_s[0, 0] * invB,
            twt_s[0, 0] * invB,
            twi_mx[0, 0],
            twi_mn[0, 0])


# speedup vs baseline: 1.4560x; 1.2001x over previous
import functools

import jax
import jax.numpy as jnp
from jax import lax
from jax.experimental import pallas as pl
from jax.experimental.pallas import tpu as pltpu

B = 2048
D = 256
R = 256
NB = B // R
GAMMA = 0.8
EPS = 1e-14
RHO = 8.0
TAU_INIT = 0.01
BETA_U = 0.5
GRAD_CLIP = 5.0
ETA_INIT = 1e-05


def _side_body(feat_ref, otherT_ref, tau_ref, s_ref, b_ref, e0_ref,
               loss_ref, twsum_ref, twmax_ref, twmin_ref, tausum_ref):
    i = pl.program_id(0)
    feat = feat_ref[...]
    otherT = otherT_ref[...]
    S = jnp.dot(feat, otherT, preferred_element_type=jnp.float32)
    row = lax.broadcasted_iota(jnp.int32, (R, B), 0)
    col = lax.broadcasted_iota(jnp.int32, (R, B), 1)
    is_diag = col == (i * R + row)
    d = jnp.sum(jnp.where(is_diag, S, 0.0), axis=1)
    tau = tau_ref[0, 0, :]
    s_old = s_ref[0, 0, :]
    b_old = b_ref[0, 0, :]
    diffs = S - d[:, None]
    idt = diffs / tau[:, None]
    b_new = jnp.maximum(b_old, jnp.max(idt, axis=1))
    E = jnp.exp(idt - b_new[:, None])
    E = jnp.where(is_diag, 0.0, E)
    g = jnp.sum(E, axis=1)
    ema = (1.0 - GAMMA) * s_old * jnp.exp(b_old - b_new) + GAMMA * g
    e0 = e0_ref[0, 0]
    sI = e0 * g + (1.0 - e0) * ema
    sIc = jnp.maximum(sI, EPS)
    w = E / sIc[:, None]
    loss_rows = jnp.sum(w * diffs, axis=1)
    wid_rows = jnp.sum(w * idt, axis=1)
    tw = jnp.log(sIc / (B - 1)) + b_new + RHO - wid_rows
    tw = jnp.clip(tw, -GRAD_CLIP, GRAD_CLIP)

    blk_loss = jnp.sum(loss_rows)
    blk_twsum = jnp.sum(tw)
    blk_twmax = jnp.max(tw)
    blk_twmin = jnp.min(tw)
    blk_tau = jnp.sum(tau)

    @pl.when(i == 0)
    def _init():
        loss_ref[0, 0] = blk_loss
        twsum_ref[0, 0] = blk_twsum
        twmax_ref[0, 0] = blk_twmax
        twmin_ref[0, 0] = blk_twmin
        tausum_ref[0, 0] = blk_tau

    @pl.when(i != 0)
    def _acc():
        loss_ref[0, 0] += blk_loss
        twsum_ref[0, 0] += blk_twsum
        twmax_ref[0, 0] = jnp.maximum(twmax_ref[0, 0], blk_twmax)
        twmin_ref[0, 0] = jnp.minimum(twmin_ref[0, 0], blk_twmin)
        tausum_ref[0, 0] += blk_tau


_scal = jax.ShapeDtypeStruct((1, 1), jnp.float32)


def _side(feat, otherT, tau_g, s_g, b_g, e0, interpret=False):
    tau3 = tau_g.reshape(NB, 1, R)
    s3 = s_g.reshape(NB, 1, R)
    b3 = b_g.reshape(NB, 1, R)
    smem = pltpu.MemorySpace.SMEM
    out = pl.pallas_call(
        _side_body,
        grid=(NB,),
        in_specs=[
            pl.BlockSpec((R, D), lambda i: (i, 0)),
            pl.BlockSpec((D, B), lambda i: (0, 0)),
            pl.BlockSpec((1, 1, R), lambda i: (i, 0, 0)),
            pl.BlockSpec((1, 1, R), lambda i: (i, 0, 0)),
            pl.BlockSpec((1, 1, R), lambda i: (i, 0, 0)),
            pl.BlockSpec(memory_space=smem),
        ],
        out_specs=[pl.BlockSpec((1, 1), lambda i: (0, 0), memory_space=smem)] * 5,
        out_shape=[_scal] * 5,
        interpret=interpret,
    )(feat, otherT, tau3, s3, b3, e0)
    return out


def kernel(image_features, text_features, image_ids, text_ids, epoch, max_epoch,
           s_I, s_T, b_I, b_T, u_I, u_T, tau_I, tau_T, mask_neg):
    tau_i = tau_I[image_ids]
    s_i = s_I[image_ids]
    b_i = b_I[image_ids]
    tau_t = tau_T[text_ids]
    s_t = s_T[text_ids]
    b_t = b_T[text_ids]

    e0 = (jnp.asarray(epoch) == 0).astype(jnp.float32).reshape(1, 1)
    txtT = text_features.T
    imgT = image_features.T

    li, twi_s, twi_mx, twi_mn, tau_si = _side(image_features, txtT, tau_i, s_i, b_i, e0)
    lt, twt_s, _twt_mx, _twt_mn, tau_st = _side(text_features, imgT, tau_t, s_t, b_t, e0)

    invB = jnp.float32(1.0 / B)
    total_loss = (li[0, 0] + lt[0, 0]) * invB
    return (total_loss,
            tau_si[0, 0] * invB,
            tau_st[0, 0] * invB,
            jnp.float32(ETA_INIT),
            twi_s[0, 0] * invB,
            twt_s[0, 0] * invB,
            twi_mx[0, 0],
            twi_mn[0, 0])


# speedup vs baseline: 1.9802x; 1.3600x over previous
import functools

import jax
import jax.numpy as jnp
from jax import lax
from jax.experimental import pallas as pl
from jax.experimental.pallas import tpu as pltpu
from jax.experimental.pallas import tpu_sc as plsc

B = 2048
D = 256
R = 256
NB = B // R
GAMMA = 0.8
EPS = 1e-14
RHO = 8.0
TAU_INIT = 0.01
BETA_U = 0.5
GRAD_CLIP = 5.0
ETA_INIT = 1e-05


def _side_body(feat_ref, otherT_ref, tau_ref, s_ref, b_ref, e0_ref,
               loss_ref, twsum_ref, twmax_ref, twmin_ref, tausum_ref):
    i = pl.program_id(0)
    feat = feat_ref[...]
    otherT = otherT_ref[...]
    S = jnp.dot(feat, otherT, preferred_element_type=jnp.float32)
    row = lax.broadcasted_iota(jnp.int32, (R, B), 0)
    col = lax.broadcasted_iota(jnp.int32, (R, B), 1)
    is_diag = col == (i * R + row)
    d = jnp.sum(jnp.where(is_diag, S, 0.0), axis=1)
    tau = tau_ref[0, 0, :]
    s_old = s_ref[0, 0, :]
    b_old = b_ref[0, 0, :]
    diffs = S - d[:, None]
    idt = diffs / tau[:, None]
    b_new = jnp.maximum(b_old, jnp.max(idt, axis=1))
    E = jnp.exp(idt - b_new[:, None])
    E = jnp.where(is_diag, 0.0, E)
    g = jnp.sum(E, axis=1)
    ema = (1.0 - GAMMA) * s_old * jnp.exp(b_old - b_new) + GAMMA * g
    e0 = e0_ref[0, 0]
    sI = e0 * g + (1.0 - e0) * ema
    sIc = jnp.maximum(sI, EPS)
    w = E / sIc[:, None]
    loss_rows = jnp.sum(w * diffs, axis=1)
    wid_rows = jnp.sum(w * idt, axis=1)
    tw = jnp.log(sIc / (B - 1)) + b_new + RHO - wid_rows
    tw = jnp.clip(tw, -GRAD_CLIP, GRAD_CLIP)

    blk_loss = jnp.sum(loss_rows)
    blk_twsum = jnp.sum(tw)
    blk_twmax = jnp.max(tw)
    blk_twmin = jnp.min(tw)
    blk_tau = jnp.sum(tau)

    @pl.when(i == 0)
    def _init():
        loss_ref[0, 0] = blk_loss
        twsum_ref[0, 0] = blk_twsum
        twmax_ref[0, 0] = blk_twmax
        twmin_ref[0, 0] = blk_twmin
        tausum_ref[0, 0] = blk_tau

    @pl.when(i != 0)
    def _acc():
        loss_ref[0, 0] += blk_loss
        twsum_ref[0, 0] += blk_twsum
        twmax_ref[0, 0] = jnp.maximum(twmax_ref[0, 0], blk_twmax)
        twmin_ref[0, 0] = jnp.minimum(twmin_ref[0, 0], blk_twmin)
        tausum_ref[0, 0] += blk_tau


_scal = jax.ShapeDtypeStruct((1, 1), jnp.float32)

_SC_INFO = plsc.get_sparse_core_info()
_NC, _NS = _SC_INFO.num_cores, _SC_INFO.num_subcores
_NW = _NC * _NS
_BPW = B // _NW

_vecf = jax.ShapeDtypeStruct((B,), jnp.float32)


@functools.partial(
    pl.kernel,
    mesh=plsc.VectorSubcoreMesh(core_axis_name="c", subcore_axis_name="s"),
    out_type=[_vecf] * 6,
    scratch_types=[
        pltpu.VMEM((_BPW,), jnp.int32),
        pltpu.VMEM((_BPW,), jnp.int32),
        pltpu.VMEM((_BPW,), jnp.float32),
        pltpu.SemaphoreType.DMA,
    ],
)
def _gather6(img_ids, txt_ids, tau_i_t, s_i_t, b_i_t, tau_t_t, s_t_t, b_t_t,
             o_tau_i, o_s_i, o_b_i, o_tau_t, o_s_t, o_b_t,
             idx_i, idx_t, buf, sem):
    wid = lax.axis_index("s") * _NC + lax.axis_index("c")
    base = wid * _BPW
    pltpu.sync_copy(img_ids.at[pl.ds(base, _BPW)], idx_i)
    pltpu.sync_copy(txt_ids.at[pl.ds(base, _BPW)], idx_t)
    for table, idx, out in ((tau_i_t, idx_i, o_tau_i), (s_i_t, idx_i, o_s_i),
                            (b_i_t, idx_i, o_b_i), (tau_t_t, idx_t, o_tau_t),
                            (s_t_t, idx_t, o_s_t), (b_t_t, idx_t, o_b_t)):
        pltpu.async_copy(table.at[idx], buf, sem).wait()
        pltpu.sync_copy(buf, out.at[pl.ds(base, _BPW)])


def _side(feat, otherT, tau_g, s_g, b_g, e0, interpret=False):
    tau3 = tau_g.reshape(NB, 1, R)
    s3 = s_g.reshape(NB, 1, R)
    b3 = b_g.reshape(NB, 1, R)
    smem = pltpu.MemorySpace.SMEM
    out = pl.pallas_call(
        _side_body,
        grid=(NB,),
        in_specs=[
            pl.BlockSpec((R, D), lambda i: (i, 0)),
            pl.BlockSpec((D, B), lambda i: (0, 0)),
            pl.BlockSpec((1, 1, R), lambda i: (i, 0, 0)),
            pl.BlockSpec((1, 1, R), lambda i: (i, 0, 0)),
            pl.BlockSpec((1, 1, R), lambda i: (i, 0, 0)),
            pl.BlockSpec(memory_space=smem),
        ],
        out_specs=[pl.BlockSpec((1, 1), lambda i: (0, 0), memory_space=smem)] * 5,
        out_shape=[_scal] * 5,
        interpret=interpret,
    )(feat, otherT, tau3, s3, b3, e0)
    return out


def kernel(image_features, text_features, image_ids, text_ids, epoch, max_epoch,
           s_I, s_T, b_I, b_T, u_I, u_T, tau_I, tau_T, mask_neg):
    tau_i, s_i, b_i, tau_t, s_t, b_t = _gather6(
        image_ids, text_ids, tau_I, s_I, b_I, tau_T, s_T, b_T)

    e0 = (jnp.asarray(epoch) == 0).astype(jnp.float32).reshape(1, 1)
    txtT = text_features.T
    imgT = image_features.T

    li, twi_s, twi_mx, twi_mn, tau_si = _side(image_features, txtT, tau_i, s_i, b_i, e0)
    lt, twt_s, _twt_mx, _twt_mn, tau_st = _side(text_features, imgT, tau_t, s_t, b_t, e0)

    invB = jnp.float32(1.0 / B)
    total_loss = (li[0, 0] + lt[0, 0]) * invB
    return (total_loss,
            tau_si[0, 0] * invB,
            tau_st[0, 0] * invB,
            jnp.float32(ETA_INIT),
            twi_s[0, 0] * invB,
            twt_s[0, 0] * invB,
            twi_mx[0, 0],
            twi_mn[0, 0])


# speedup vs baseline: 2.0809x; 1.0508x over previous
import functools

import jax
import jax.numpy as jnp
from jax import lax
from jax.experimental import pallas as pl
from jax.experimental.pallas import tpu as pltpu
from jax.experimental.pallas import tpu_sc as plsc

B = 2048
D = 256
R = 256
NB = B // R
GAMMA = 0.8
EPS = 1e-14
RHO = 8.0
TAU_INIT = 0.01
BETA_U = 0.5
GRAD_CLIP = 5.0
ETA_INIT = 1e-05


def _side_body(feat_ref, otherT_ref, tau_ref, s_ref, b_ref, e0_ref,
               loss_ref, twsum_ref, twmax_ref, twmin_ref, tausum_ref):
    i = pl.program_id(0)
    feat = feat_ref[...]
    otherT = otherT_ref[...]
    S = jnp.dot(feat, otherT, preferred_element_type=jnp.float32)
    row = lax.broadcasted_iota(jnp.int32, (R, B), 0)
    col = lax.broadcasted_iota(jnp.int32, (R, B), 1)
    is_diag = col == (i * R + row)
    d = jnp.sum(jnp.where(is_diag, S, 0.0), axis=1)
    tau = tau_ref[0, 0, :]
    s_old = s_ref[0, 0, :]
    b_old = b_ref[0, 0, :]
    rtau = 1.0 / tau
    diffs = S - d[:, None]
    idt = diffs * rtau[:, None]
    b_new = jnp.maximum(b_old, jnp.max(idt, axis=1))
    E = jnp.exp(idt - b_new[:, None])
    E = jnp.where(is_diag, 0.0, E)
    g = jnp.sum(E, axis=1)
    ema = (1.0 - GAMMA) * s_old * jnp.exp(b_old - b_new) + GAMMA * g
    e0 = e0_ref[0, 0]
    sI = e0 * g + (1.0 - e0) * ema
    sIc = jnp.maximum(sI, EPS)
    P1 = jnp.sum(E * diffs, axis=1)
    rs = 1.0 / sIc
    loss_rows = P1 * rs
    wid_rows = loss_rows * rtau
    tw = jnp.log(sIc / (B - 1)) + b_new + RHO - wid_rows
    tw = jnp.clip(tw, -GRAD_CLIP, GRAD_CLIP)

    blk_loss = jnp.sum(loss_rows)
    blk_twsum = jnp.sum(tw)
    blk_twmax = jnp.max(tw)
    blk_twmin = jnp.min(tw)
    blk_tau = jnp.sum(tau)

    @pl.when(i == 0)
    def _init():
        loss_ref[0, 0] = blk_loss
        twsum_ref[0, 0] = blk_twsum
        twmax_ref[0, 0] = blk_twmax
        twmin_ref[0, 0] = blk_twmin
        tausum_ref[0, 0] = blk_tau

    @pl.when(i != 0)
    def _acc():
        loss_ref[0, 0] += blk_loss
        twsum_ref[0, 0] += blk_twsum
        twmax_ref[0, 0] = jnp.maximum(twmax_ref[0, 0], blk_twmax)
        twmin_ref[0, 0] = jnp.minimum(twmin_ref[0, 0], blk_twmin)
        tausum_ref[0, 0] += blk_tau


_scal = jax.ShapeDtypeStruct((1, 1), jnp.float32)

_SC_INFO = plsc.get_sparse_core_info()
_NC, _NS = _SC_INFO.num_cores, _SC_INFO.num_subcores
_NW = _NC * _NS
_BPW = B // _NW

_vecf = jax.ShapeDtypeStruct((B,), jnp.float32)


@functools.partial(
    pl.kernel,
    mesh=plsc.VectorSubcoreMesh(core_axis_name="c", subcore_axis_name="s"),
    out_type=[_vecf] * 6,
    scratch_types=[
        pltpu.VMEM((_BPW,), jnp.int32),
        pltpu.VMEM((_BPW,), jnp.int32),
        pltpu.VMEM((_BPW,), jnp.float32),
        pltpu.SemaphoreType.DMA,
    ],
)
def _gather6(img_ids, txt_ids, tau_i_t, s_i_t, b_i_t, tau_t_t, s_t_t, b_t_t,
             o_tau_i, o_s_i, o_b_i, o_tau_t, o_s_t, o_b_t,
             idx_i, idx_t, buf, sem):
    wid = lax.axis_index("s") * _NC + lax.axis_index("c")
    base = wid * _BPW
    pltpu.sync_copy(img_ids.at[pl.ds(base, _BPW)], idx_i)
    pltpu.sync_copy(txt_ids.at[pl.ds(base, _BPW)], idx_t)
    for table, idx, out in ((tau_i_t, idx_i, o_tau_i), (s_i_t, idx_i, o_s_i),
                            (b_i_t, idx_i, o_b_i), (tau_t_t, idx_t, o_tau_t),
                            (s_t_t, idx_t, o_s_t), (b_t_t, idx_t, o_b_t)):
        pltpu.async_copy(table.at[idx], buf, sem).wait()
        pltpu.sync_copy(buf, out.at[pl.ds(base, _BPW)])


def _side(feat, otherT, tau_g, s_g, b_g, e0, interpret=False):
    tau3 = tau_g.reshape(NB, 1, R)
    s3 = s_g.reshape(NB, 1, R)
    b3 = b_g.reshape(NB, 1, R)
    smem = pltpu.MemorySpace.SMEM
    out = pl.pallas_call(
        _side_body,
        grid=(NB,),
        in_specs=[
            pl.BlockSpec((R, D), lambda i: (i, 0)),
            pl.BlockSpec((D, B), lambda i: (0, 0)),
            pl.BlockSpec((1, 1, R), lambda i: (i, 0, 0)),
            pl.BlockSpec((1, 1, R), lambda i: (i, 0, 0)),
            pl.BlockSpec((1, 1, R), lambda i: (i, 0, 0)),
            pl.BlockSpec(memory_space=smem),
        ],
        out_specs=[pl.BlockSpec((1, 1), lambda i: (0, 0), memory_space=smem)] * 5,
        out_shape=[_scal] * 5,
        interpret=interpret,
    )(feat, otherT, tau3, s3, b3, e0)
    return out


def kernel(image_features, text_features, image_ids, text_ids, epoch, max_epoch,
           s_I, s_T, b_I, b_T, u_I, u_T, tau_I, tau_T, mask_neg):
    tau_i, s_i, b_i, tau_t, s_t, b_t = _gather6(
        image_ids, text_ids, tau_I, s_I, b_I, tau_T, s_T, b_T)

    e0 = (jnp.asarray(epoch) == 0).astype(jnp.float32).reshape(1, 1)
    txtT = text_features.T
    imgT = image_features.T

    li, twi_s, twi_mx, twi_mn, tau_si = _side(image_features, txtT, tau_i, s_i, b_i, e0)
    lt, twt_s, _twt_mx, _twt_mn, tau_st = _side(text_features, imgT, tau_t, s_t, b_t, e0)

    invB = jnp.float32(1.0 / B)
    total_loss = (li[0, 0] + lt[0, 0]) * invB
    return (total_loss,
            tau_si[0, 0] * invB,
            tau_st[0, 0] * invB,
            jnp.float32(ETA_INIT),
            twi_s[0, 0] * invB,
            twt_s[0, 0] * invB,
            twi_mx[0, 0],
            twi_mn[0, 0])


# speedup vs baseline: 2.1916x; 1.0532x over previous
import functools

import jax
import jax.numpy as jnp
from jax import lax
from jax.experimental import pallas as pl
from jax.experimental.pallas import tpu as pltpu
from jax.experimental.pallas import tpu_sc as plsc

B = 2048
D = 256
R = 256
NB = B // R
GAMMA = 0.8
EPS = 1e-14
RHO = 8.0
TAU_INIT = 0.01
BETA_U = 0.5
GRAD_CLIP = 5.0
ETA_INIT = 1e-05


def _side_body(feat_ref, other_ref, orows_ref, tau_ref, s_ref, b_ref, e0_ref,
               loss_ref, twsum_ref, twmax_ref, twmin_ref, tausum_ref):
    i = pl.program_id(0)
    feat = feat_ref[...]
    other = other_ref[...]
    S = lax.dot_general(feat, other, (((1,), (1,)), ((), ())),
                        preferred_element_type=jnp.float32)
    d = jnp.sum(feat * orows_ref[...], axis=1)
    tau = tau_ref[0, 0, :]
    s_old = s_ref[0, 0, :]
    b_old = b_ref[0, 0, :]
    rtau = 1.0 / tau
    m = jnp.max(S, axis=1)
    b_new = jnp.maximum(b_old, (m - d) * rtau)
    c2 = d * rtau + b_new
    E = jnp.exp(S * rtau[:, None] - c2[:, None])
    gwd = jnp.sum(E, axis=1)
    sES = jnp.sum(E * S, axis=1)
    g = gwd - jnp.exp(-b_new)
    P1 = sES - d * gwd
    ema = (1.0 - GAMMA) * s_old * jnp.exp(b_old - b_new) + GAMMA * g
    e0 = e0_ref[0, 0]
    sI = e0 * g + (1.0 - e0) * ema
    sIc = jnp.maximum(sI, EPS)
    rs = 1.0 / sIc
    loss_rows = P1 * rs
    wid_rows = loss_rows * rtau
    tw = jnp.log(sIc / (B - 1)) + b_new + RHO - wid_rows
    tw = jnp.clip(tw, -GRAD_CLIP, GRAD_CLIP)

    blk_loss = jnp.sum(loss_rows)
    blk_twsum = jnp.sum(tw)
    blk_twmax = jnp.max(tw)
    blk_twmin = jnp.min(tw)
    blk_tau = jnp.sum(tau)

    @pl.when(i == 0)
    def _init():
        loss_ref[0, 0] = blk_loss
        twsum_ref[0, 0] = blk_twsum
        twmax_ref[0, 0] = blk_twmax
        twmin_ref[0, 0] = blk_twmin
        tausum_ref[0, 0] = blk_tau

    @pl.when(i != 0)
    def _acc():
        loss_ref[0, 0] += blk_loss
        twsum_ref[0, 0] += blk_twsum
        twmax_ref[0, 0] = jnp.maximum(twmax_ref[0, 0], blk_twmax)
        twmin_ref[0, 0] = jnp.minimum(twmin_ref[0, 0], blk_twmin)
        tausum_ref[0, 0] += blk_tau


_scal = jax.ShapeDtypeStruct((1, 1), jnp.float32)

_SC_INFO = plsc.get_sparse_core_info()
_NC, _NS = _SC_INFO.num_cores, _SC_INFO.num_subcores
_NW = _NC * _NS
_BPW = B // _NW

_vecf = jax.ShapeDtypeStruct((B,), jnp.float32)


@functools.partial(
    pl.kernel,
    mesh=plsc.VectorSubcoreMesh(core_axis_name="c", subcore_axis_name="s"),
    out_type=[_vecf] * 6,
    scratch_types=[
        pltpu.VMEM((_BPW,), jnp.int32),
        pltpu.VMEM((_BPW,), jnp.int32),
        pltpu.VMEM((_BPW,), jnp.float32),
        pltpu.SemaphoreType.DMA,
    ],
)
def _gather6(img_ids, txt_ids, tau_i_t, s_i_t, b_i_t, tau_t_t, s_t_t, b_t_t,
             o_tau_i, o_s_i, o_b_i, o_tau_t, o_s_t, o_b_t,
             idx_i, idx_t, buf, sem):
    wid = lax.axis_index("s") * _NC + lax.axis_index("c")
    base = wid * _BPW
    pltpu.sync_copy(img_ids.at[pl.ds(base, _BPW)], idx_i)
    pltpu.sync_copy(txt_ids.at[pl.ds(base, _BPW)], idx_t)
    for table, idx, out in ((tau_i_t, idx_i, o_tau_i), (s_i_t, idx_i, o_s_i),
                            (b_i_t, idx_i, o_b_i), (tau_t_t, idx_t, o_tau_t),
                            (s_t_t, idx_t, o_s_t), (b_t_t, idx_t, o_b_t)):
        pltpu.async_copy(table.at[idx], buf, sem).wait()
        pltpu.sync_copy(buf, out.at[pl.ds(base, _BPW)])


def _side(feat, other, tau_g, s_g, b_g, e0, interpret=False):
    tau3 = tau_g.reshape(NB, 1, R)
    s3 = s_g.reshape(NB, 1, R)
    b3 = b_g.reshape(NB, 1, R)
    smem = pltpu.MemorySpace.SMEM
    out = pl.pallas_call(
        _side_body,
        grid=(NB,),
        in_specs=[
            pl.BlockSpec((R, D), lambda i: (i, 0)),
            pl.BlockSpec((B, D), lambda i: (0, 0)),
            pl.BlockSpec((R, D), lambda i: (i, 0)),
            pl.BlockSpec((1, 1, R), lambda i: (i, 0, 0)),
            pl.BlockSpec((1, 1, R), lambda i: (i, 0, 0)),
            pl.BlockSpec((1, 1, R), lambda i: (i, 0, 0)),
            pl.BlockSpec(memory_space=smem),
        ],
        out_specs=[pl.BlockSpec((1, 1), lambda i: (0, 0), memory_space=smem)] * 5,
        out_shape=[_scal] * 5,
        interpret=interpret,
    )(feat, other, other, tau3, s3, b3, e0)
    return out


def kernel(image_features, text_features, image_ids, text_ids, epoch, max_epoch,
           s_I, s_T, b_I, b_T, u_I, u_T, tau_I, tau_T, mask_neg):
    tau_i, s_i, b_i, tau_t, s_t, b_t = _gather6(
        image_ids, text_ids, tau_I, s_I, b_I, tau_T, s_T, b_T)

    e0 = (jnp.asarray(epoch) == 0).astype(jnp.float32).reshape(1, 1)

    li, twi_s, twi_mx, twi_mn, tau_si = _side(image_features, text_features,
                                              tau_i, s_i, b_i, e0)
    lt, twt_s, _twt_mx, _twt_mn, tau_st = _side(text_features, image_features,
                                                tau_t, s_t, b_t, e0)

    invB = jnp.float32(1.0 / B)
    total_loss = (li[0, 0] + lt[0, 0]) * invB
    return (total_loss,
            tau_si[0, 0] * invB,
            tau_st[0, 0] * invB,
            jnp.float32(ETA_INIT),
            twi_s[0, 0] * invB,
            twt_s[0, 0] * invB,
            twi_mx[0, 0],
            twi_mn[0, 0])
